# baseline (speedup 1.0000x reference)
import functools

import jax
import jax.numpy as jnp
from jax.experimental import pallas as pl
from jax.experimental.pallas import tpu as pltpu

_BS = 8
_L = 64
_IMG = 224
_K = 16
_BIG_I = 2 ** 30
_NL = 8


def _one_cell(pred_ref, img_ref, out_ref, b, l, consts):
    lane128, kiota, lin = consts

    n = l * _BS + b
    b2 = n // _L
    l2 = n - b2 * _L
    px = pred_ref[b2, l2, 0]
    py = pred_ref[b2, l2, 1]
    gx = 2.0 * px - 1.0
    gy = 2.0 * py - 1.0
    fx = ((gx + 1.0) * _IMG - 1.0) / 2.0
    fy = ((gy + 1.0) * _IMG - 1.0) / 2.0

    def _round_i32(v):
        r = jnp.round(jnp.full((1, 128), v, jnp.float32)).astype(jnp.int32)
        return jnp.max(r)

    ixn = _round_i32(fx)
    iyn = _round_i32(fy)
    valid = ((ixn >= 0) & (ixn < _IMG) & (iyn >= 0) & (iyn < _IMG))
    vf = valid.astype(jnp.float32)
    ixc = jnp.clip(ixn, 0, _IMG - 1)
    iyc = jnp.clip(iyn, 0, _IMG - 1)

    lane = jax.lax.broadcasted_iota(jnp.int32, (1, _IMG), 1)
    sel = (lane == ixc).astype(jnp.float32)
    pooled = []
    for c in range(3):
        row = img_ref[0, c, pl.ds(iyc, 1), :]
        pooled.append(jnp.sum(row * sel) * vf)

    d = (img_ref[0, 0] - pooled[0]) ** 2
    d = d + (img_ref[0, 1] - pooled[1]) ** 2
    d = d + (img_ref[0, 2] - pooled[2]) ** 2

    bits = jax.lax.bitcast_convert_type(d, jnp.int32)
    inf_bits = 0x7F800000
    xs = jnp.zeros((1, _K), jnp.float32)
    ys = jnp.zeros((1, _K), jnp.float32)
    for k in range(_K):
        m = jnp.min(bits)
        pmin = jnp.min(jnp.where(bits == m, lin, _BIG_I))
        bits = jnp.where(lin == pmin, inf_bits, bits)
        prow = pmin // _IMG
        pcol = pmin - prow * _IMG
        xk = pcol.astype(jnp.float32) / _IMG
        yk = prow.astype(jnp.float32) / _IMG
        hit = kiota == k
        xs = jnp.where(hit, xk, xs)
        ys = jnp.where(hit, yk, ys)

    @pl.when(l < _L - 1)
    def _():
        qx = pred_ref[b, l + 1, 0]
        qy = pred_ref[b, l + 1, 1]
        dk = (qx - xs) ** 2 + (qy - ys) ** 2
        dbest = jnp.min(dk)
        kbest = jnp.min(jnp.where(dk == dbest, kiota, _BIG_I))
        hitk = kiota == kbest
        bx = jnp.sum(jnp.where(hitk, xs, 0.0))
        by = jnp.sum(jnp.where(hitk, ys, 0.0))
        out_ref[b, l + 1] = (qx - bx) ** 2 + (qy - by) ** 2

    @pl.when(l == _L - 1)
    def _():
        out_ref[b, 0] = 0.0


def _ccloss_kernel(pred_ref, img_ref, out_ref):
    b = pl.program_id(0)
    l0 = pl.program_id(1)
    lane128 = jax.lax.broadcasted_iota(jnp.int32, (1, 128), 1)
    kiota = jax.lax.broadcasted_iota(jnp.int32, (1, _K), 1)
    rows = jax.lax.broadcasted_iota(jnp.int32, (_IMG, _IMG), 0)
    cols = jax.lax.broadcasted_iota(jnp.int32, (_IMG, _IMG), 1)
    lin = rows * _IMG + cols
    consts = (lane128, kiota, lin)
    for j in range(_NL):
        _one_cell(pred_ref, img_ref, out_ref, b, l0 * _NL + j, consts)


@jax.jit
def kernel(predictions, ref_imgs):
    contrib = pl.pallas_call(
        _ccloss_kernel,
        grid=(_BS, _L // _NL),
        in_specs=[
            pl.BlockSpec(memory_space=pltpu.SMEM),
            pl.BlockSpec((1, 3, _IMG, _IMG), lambda b, l: (b, 0, 0, 0)),
        ],
        out_specs=pl.BlockSpec(memory_space=pltpu.SMEM),
        out_shape=jax.ShapeDtypeStruct((_BS, _L), jnp.float32),
    )(predictions, ref_imgs)
    return jnp.mean(contrib[:, 1:])


# speedup vs baseline: 5.1267x; 5.1267x over previous
import functools

import jax
import jax.numpy as jnp
from jax.experimental import pallas as pl
from jax.experimental.pallas import tpu as pltpu

_BS = 8
_L = 64
_IMG = 224
_R = 392
_K = 16
_BIG_I = 2 ** 30
_INF_BITS = 0x7F800000
_NB = 8


def _ccloss_kernel(pred_ref, img_ref, out_ref):
    b = pl.program_id(0)
    g = pl.program_id(1)

    lane = jax.lax.broadcasted_iota(jnp.int32, (1, 128), 1)
    kiota = jax.lax.broadcasted_iota(jnp.int32, (1, _K), 1)
    rows = jax.lax.broadcasted_iota(jnp.int32, (1, _R, 128), 1)
    cols = jax.lax.broadcasted_iota(jnp.int32, (1, _R, 128), 2)
    lin = rows * 128 + cols
    biota = jax.lax.broadcasted_iota(jnp.int32, (_NB, 1, 1), 0)

    def _round_i32(v):
        r = jnp.round(jnp.full((1, 128), v, jnp.float32)).astype(jnp.int32)
        return jnp.max(r)

    pooled = [jnp.zeros((_NB, 1, 1), jnp.float32) for _ in range(3)]
    for j in range(_NB):
        l = g * _NB + j
        n = l * _BS + b
        b2 = n // _L
        l2 = n - b2 * _L
        px = pred_ref[b2, l2, 0]
        py = pred_ref[b2, l2, 1]
        fx = ((2.0 * px - 1.0 + 1.0) * _IMG - 1.0) / 2.0
        fy = ((2.0 * py - 1.0 + 1.0) * _IMG - 1.0) / 2.0
        ixn = _round_i32(fx)
        iyn = _round_i32(fy)
        valid = ((ixn >= 0) & (ixn < _IMG) & (iyn >= 0) & (iyn < _IMG))
        vf = valid.astype(jnp.float32)
        ixc = jnp.clip(ixn, 0, _IMG - 1)
        iyc = jnp.clip(iyn, 0, _IMG - 1)
        p = iyc * _IMG + ixc
        pr = p // 128
        pc = p - pr * 128
        sel = (lane == pc).astype(jnp.float32)
        hit = biota == j
        for c in range(3):
            row = img_ref[0, c, pl.ds(pr, 1), :]
            s = jnp.sum(row * sel) * vf
            pooled[c] = jnp.where(hit, s, pooled[c])

    d = (img_ref[0, 0][None] - pooled[0]) ** 2
    d = d + (img_ref[0, 1][None] - pooled[1]) ** 2
    d = d + (img_ref[0, 2][None] - pooled[2]) ** 2

    bits = jax.lax.bitcast_convert_type(d, jnp.int32)
    xs = jnp.zeros((_NB, 1, _K), jnp.float32)
    ys = jnp.zeros((_NB, 1, _K), jnp.float32)
    for k in range(_K):
        m = jnp.min(jnp.min(bits, axis=1, keepdims=True), axis=2, keepdims=True)
        cand = jnp.where(bits == m, lin, _BIG_I)
        pmin = jnp.min(jnp.min(cand, axis=1, keepdims=True), axis=2,
                       keepdims=True)
        bits = jnp.where(lin == pmin, _INF_BITS, bits)
        prow = pmin // _IMG
        pcol = pmin - prow * _IMG
        xk = pcol.astype(jnp.float32) / _IMG
        yk = prow.astype(jnp.float32) / _IMG
        hitk = kiota[None] == k
        xs = jnp.where(hitk, xk, xs)
        ys = jnp.where(hitk, yk, ys)

    for j in range(_NB):
        l = g * _NB + j
        xsj = xs[j]
        ysj = ys[j]

        @pl.when(l < _L - 1)
        def _():
            qx = pred_ref[b, l + 1, 0]
            qy = pred_ref[b, l + 1, 1]
            dk = (qx - xsj) ** 2 + (qy - ysj) ** 2
            dbest = jnp.min(dk)
            kbest = jnp.min(jnp.where(dk == dbest, kiota, _BIG_I))
            hitk = kiota == kbest
            bx = jnp.sum(jnp.where(hitk, xsj, 0.0))
            by = jnp.sum(jnp.where(hitk, ysj, 0.0))
            out_ref[b, l + 1] = (qx - bx) ** 2 + (qy - by) ** 2

        @pl.when(l == _L - 1)
        def _():
            out_ref[b, 0] = 0.0


@jax.jit
def kernel(predictions, ref_imgs):
    imgs2 = ref_imgs.reshape(_BS, 3, _R, 128)
    contrib = pl.pallas_call(
        _ccloss_kernel,
        grid=(_BS, _L // _NB),
        in_specs=[
            pl.BlockSpec(memory_space=pltpu.SMEM),
            pl.BlockSpec((1, 3, _R, 128), lambda b, g: (b, 0, 0, 0)),
        ],
        out_specs=pl.BlockSpec(memory_space=pltpu.SMEM),
        out_shape=jax.ShapeDtypeStruct((_BS, _L), jnp.float32),
    )(predictions, imgs2)
    return jnp.mean(contrib[:, 1:])


# speedup vs baseline: 5.2189x; 1.0180x over previous
import functools

import jax
import jax.numpy as jnp
from jax.experimental import pallas as pl
from jax.experimental.pallas import tpu as pltpu

_BS = 8
_L = 64
_IMG = 224
_R = 392
_K = 16
_BIG_I = 2 ** 30
_INF_BITS = 0x7F800000
_NB = 16


def _ccloss_kernel(pred_ref, img_ref, out_ref):
    b = pl.program_id(0)
    g = pl.program_id(1)

    lane = jax.lax.broadcasted_iota(jnp.int32, (1, 128), 1)
    kiota = jax.lax.broadcasted_iota(jnp.int32, (1, _K), 1)
    rows = jax.lax.broadcasted_iota(jnp.int32, (1, _R, 128), 1)
    cols = jax.lax.broadcasted_iota(jnp.int32, (1, _R, 128), 2)
    lin = rows * 128 + cols
    biota = jax.lax.broadcasted_iota(jnp.int32, (_NB, 1, 1), 0)

    def _round_i32(v):
        r = jnp.round(jnp.full((1, 128), v, jnp.float32)).astype(jnp.int32)
        return jnp.max(r)

    pooled = [jnp.zeros((_NB, 1, 1), jnp.float32) for _ in range(3)]
    for j in range(_NB):
        l = g * _NB + j
        n = l * _BS + b
        b2 = n // _L
        l2 = n - b2 * _L
        px = pred_ref[b2, l2, 0]
        py = pred_ref[b2, l2, 1]
        fx = ((2.0 * px - 1.0 + 1.0) * _IMG - 1.0) / 2.0
        fy = ((2.0 * py - 1.0 + 1.0) * _IMG - 1.0) / 2.0
        ixn = _round_i32(fx)
        iyn = _round_i32(fy)
        valid = ((ixn >= 0) & (ixn < _IMG) & (iyn >= 0) & (iyn < _IMG))
        vf = valid.astype(jnp.float32)
        ixc = jnp.clip(ixn, 0, _IMG - 1)
        iyc = jnp.clip(iyn, 0, _IMG - 1)
        p = iyc * _IMG + ixc
        pr = p // 128
        pc = p - pr * 128
        sel = (lane == pc).astype(jnp.float32)
        hit = biota == j
        for c in range(3):
            row = img_ref[0, c, pl.ds(pr, 1), :]
            s = jnp.sum(row * sel) * vf
            pooled[c] = jnp.where(hit, s, pooled[c])

    d = (img_ref[0, 0][None] - pooled[0]) ** 2
    d = d + (img_ref[0, 1][None] - pooled[1]) ** 2
    d = d + (img_ref[0, 2][None] - pooled[2]) ** 2

    bits = jax.lax.bitcast_convert_type(d, jnp.int32)
    xs = jnp.zeros((_NB, 1, _K), jnp.float32)
    ys = jnp.zeros((_NB, 1, _K), jnp.float32)
    for k in range(_K):
        m = jnp.min(jnp.min(bits, axis=1, keepdims=True), axis=2, keepdims=True)
        cand = jnp.where(bits == m, lin, _BIG_I)
        pmin = jnp.min(jnp.min(cand, axis=1, keepdims=True), axis=2,
                       keepdims=True)
        bits = jnp.where(lin == pmin, _INF_BITS, bits)
        prow = pmin // _IMG
        pcol = pmin - prow * _IMG
        xk = pcol.astype(jnp.float32) / _IMG
        yk = prow.astype(jnp.float32) / _IMG
        hitk = kiota[None] == k
        xs = jnp.where(hitk, xk, xs)
        ys = jnp.where(hitk, yk, ys)

    for j in range(_NB):
        l = g * _NB + j
        xsj = xs[j]
        ysj = ys[j]

        @pl.when(l < _L - 1)
        def _():
            qx = pred_ref[b, l + 1, 0]
            qy = pred_ref[b, l + 1, 1]
            dk = (qx - xsj) ** 2 + (qy - ysj) ** 2
            dbest = jnp.min(dk)
            kbest = jnp.min(jnp.where(dk == dbest, kiota, _BIG_I))
            hitk = kiota == kbest
            bx = jnp.sum(jnp.where(hitk, xsj, 0.0))
            by = jnp.sum(jnp.where(hitk, ysj, 0.0))
            out_ref[b, l + 1] = (qx - bx) ** 2 + (qy - by) ** 2

        @pl.when(l == _L - 1)
        def _():
            out_ref[b, 0] = 0.0


@jax.jit
def kernel(predictions, ref_imgs):
    imgs2 = ref_imgs.reshape(_BS, 3, _R, 128)
    contrib = pl.pallas_call(
        _ccloss_kernel,
        grid=(_BS, _L // _NB),
        in_specs=[
            pl.BlockSpec(memory_space=pltpu.SMEM),
            pl.BlockSpec((1, 3, _R, 128), lambda b, g: (b, 0, 0, 0)),
        ],
        out_specs=pl.BlockSpec(memory_space=pltpu.SMEM),
        out_shape=jax.ShapeDtypeStruct((_BS, _L), jnp.float32),
    )(predictions, imgs2)
    return jnp.mean(contrib[:, 1:])


# speedup vs baseline: 7.0569x; 1.3522x over previous
import functools

import jax
import jax.numpy as jnp
from jax.experimental import pallas as pl
from jax.experimental.pallas import tpu as pltpu

_BS = 8
_L = 64
_IMG = 224
_R = 392
_K = 16
_D = 4
_BIG_I = 2 ** 30
_INF_BITS = 0x7F800000
_NB = 16


def _ccloss_kernel(pred_ref, img_ref, out_ref):
    b = pl.program_id(0)
    g = pl.program_id(1)

    lane = jax.lax.broadcasted_iota(jnp.int32, (1, 128), 1)
    kiota = jax.lax.broadcasted_iota(jnp.int32, (1, _K), 1)
    rows3 = jax.lax.broadcasted_iota(jnp.int32, (1, _R, 128), 1)
    cols3 = jax.lax.broadcasted_iota(jnp.int32, (1, _R, 128), 2)
    lane3 = jax.lax.broadcasted_iota(jnp.int32, (_NB, 1, 128), 2)
    kio3 = jax.lax.broadcasted_iota(jnp.int32, (_NB, 1, _K), 2)
    biota = jax.lax.broadcasted_iota(jnp.int32, (_NB, 1, 1), 0)

    def _round_i32(v):
        r = jnp.round(jnp.full((1, 128), v, jnp.float32)).astype(jnp.int32)
        return jnp.max(r)

    pooled = [jnp.zeros((_NB, 1, 1), jnp.float32) for _ in range(3)]
    for j in range(_NB):
        l = g * _NB + j
        n = l * _BS + b
        b2 = n // _L
        l2 = n - b2 * _L
        px = pred_ref[b2, l2, 0]
        py = pred_ref[b2, l2, 1]
        fx = ((2.0 * px - 1.0 + 1.0) * _IMG - 1.0) / 2.0
        fy = ((2.0 * py - 1.0 + 1.0) * _IMG - 1.0) / 2.0
        ixn = _round_i32(fx)
        iyn = _round_i32(fy)
        valid = ((ixn >= 0) & (ixn < _IMG) & (iyn >= 0) & (iyn < _IMG))
        vf = valid.astype(jnp.float32)
        ixc = jnp.clip(ixn, 0, _IMG - 1)
        iyc = jnp.clip(iyn, 0, _IMG - 1)
        p = iyc * _IMG + ixc
        pr = p // 128
        pc = p - pr * 128
        sel = (lane == pc).astype(jnp.float32)
        hit = biota == j
        for c in range(3):
            row = img_ref[0, c, pl.ds(pr, 1), :]
            s = jnp.sum(row * sel) * vf
            pooled[c] = jnp.where(hit, s, pooled[c])

    d = (img_ref[0, 0][None] - pooled[0]) ** 2
    d = d + (img_ref[0, 1][None] - pooled[1]) ** 2
    d = d + (img_ref[0, 2][None] - pooled[2]) ** 2

    bits = jax.lax.bitcast_convert_type(d, jnp.int32)

    Ms, As = [], []
    w = bits
    for dpt in range(_D):
        M = jnp.min(w, axis=1, keepdims=True)
        Arow = jnp.min(jnp.where(w == M, rows3, _BIG_I), axis=1,
                       keepdims=True)
        Ms.append(M)
        As.append(Arow)
        if dpt < _D - 1:
            w = jnp.where(rows3 == Arow, _INF_BITS, w)

    Mcur, Acur = Ms[0], As[0]
    dep = jnp.zeros((_NB, 1, 128), jnp.int32)
    badv = jnp.zeros((_NB, 1, 128), jnp.int32)
    xs = jnp.zeros((_NB, 1, _K), jnp.float32)
    ys = jnp.zeros((_NB, 1, _K), jnp.float32)
    for t in range(_K):
        m = jnp.min(Mcur, axis=2, keepdims=True)
        plin = Acur * 128 + lane3
        cand = jnp.where(Mcur == m, plin, _BIG_I)
        pmin = jnp.min(cand, axis=2, keepdims=True)
        popm = cand == pmin
        prow = pmin // _IMG
        pcol = pmin - prow * _IMG
        xk = pcol.astype(jnp.float32) / _IMG
        yk = prow.astype(jnp.float32) / _IMG
        hitk = kio3 == t
        xs = jnp.where(hitk, xk, xs)
        ys = jnp.where(hitk, yk, ys)
        dep = dep + popm.astype(jnp.int32)
        if t < _K - 1:
            badv = jnp.maximum(
                badv, jnp.where(popm & (dep >= _D), 1, 0))
            mnx = jnp.full((_NB, 1, 128), _INF_BITS, jnp.int32)
            anx = jnp.zeros((_NB, 1, 128), jnp.int32)
            for dpt in range(1, _D):
                hitd = dep == dpt
                mnx = jnp.where(hitd, Ms[dpt], mnx)
                anx = jnp.where(hitd, As[dpt], anx)
            Mcur = jnp.where(popm, mnx, Mcur)
            Acur = jnp.where(popm, anx, Acur)
    anybad = jnp.max(badv)

    def _emit(xs_, ys_):
        for j in range(_NB):
            l = g * _NB + j
            xsj = xs_[j]
            ysj = ys_[j]

            @pl.when(l < _L - 1)
            def _():
                qx = pred_ref[b, l + 1, 0]
                qy = pred_ref[b, l + 1, 1]
                dk = (qx - xsj) ** 2 + (qy - ysj) ** 2
                dbest = jnp.min(dk)
                kbest = jnp.min(jnp.where(dk == dbest, kiota, _BIG_I))
                hk = kiota == kbest
                bx = jnp.sum(jnp.where(hk, xsj, 0.0))
                by = jnp.sum(jnp.where(hk, ysj, 0.0))
                out_ref[b, l + 1] = (qx - bx) ** 2 + (qy - by) ** 2

            @pl.when(l == _L - 1)
            def _():
                out_ref[b, 0] = 0.0

    @pl.when(anybad == 0)
    def _():
        _emit(xs, ys)

    @pl.when(anybad != 0)
    def _():
        lin = rows3 * 128 + cols3
        wb = bits
        xs2 = jnp.zeros((_NB, 1, _K), jnp.float32)
        ys2 = jnp.zeros((_NB, 1, _K), jnp.float32)
        for k in range(_K):
            m = jnp.min(jnp.min(wb, axis=1, keepdims=True), axis=2,
                        keepdims=True)
            cand = jnp.where(wb == m, lin, _BIG_I)
            pmin = jnp.min(jnp.min(cand, axis=1, keepdims=True), axis=2,
                           keepdims=True)
            wb = jnp.where(lin == pmin, _INF_BITS, wb)
            prow = pmin // _IMG
            pcol = pmin - prow * _IMG
            xk = pcol.astype(jnp.float32) / _IMG
            yk = prow.astype(jnp.float32) / _IMG
            hitk = kio3 == k
            xs2 = jnp.where(hitk, xk, xs2)
            ys2 = jnp.where(hitk, yk, ys2)
        _emit(xs2, ys2)


@jax.jit
def kernel(predictions, ref_imgs):
    imgs2 = ref_imgs.reshape(_BS, 3, _R, 128)
    contrib = pl.pallas_call(
        _ccloss_kernel,
        grid=(_BS, _L // _NB),
        in_specs=[
            pl.BlockSpec(memory_space=pltpu.SMEM),
            pl.BlockSpec((1, 3, _R, 128), lambda b, g: (b, 0, 0, 0)),
        ],
        out_specs=pl.BlockSpec(memory_space=pltpu.SMEM),
        out_shape=jax.ShapeDtypeStruct((_BS, _L), jnp.float32),
    )(predictions, imgs2)
    return jnp.mean(contrib[:, 1:])


# speedup vs baseline: 7.1630x; 1.0150x over previous
import functools

import jax
import jax.numpy as jnp
from jax.experimental import pallas as pl
from jax.experimental.pallas import tpu as pltpu

_BS = 8
_L = 64
_IMG = 224
_R = 392
_K = 16
_D = 4
_BIG_I = 2 ** 30
_INF_BITS = 0x7F800000
_NB = 32


def _ccloss_kernel(pred_ref, img_ref, out_ref):
    b = pl.program_id(0)
    g = pl.program_id(1)

    lane = jax.lax.broadcasted_iota(jnp.int32, (1, 128), 1)
    kiota = jax.lax.broadcasted_iota(jnp.int32, (1, _K), 1)
    rows3 = jax.lax.broadcasted_iota(jnp.int32, (1, _R, 128), 1)
    cols3 = jax.lax.broadcasted_iota(jnp.int32, (1, _R, 128), 2)
    lane3 = jax.lax.broadcasted_iota(jnp.int32, (_NB, 1, 128), 2)
    kio3 = jax.lax.broadcasted_iota(jnp.int32, (_NB, 1, _K), 2)
    biota = jax.lax.broadcasted_iota(jnp.int32, (_NB, 1, 1), 0)

    def _round_i32(v):
        r = jnp.round(jnp.full((1, 128), v, jnp.float32)).astype(jnp.int32)
        return jnp.max(r)

    pooled = [jnp.zeros((_NB, 1, 1), jnp.float32) for _ in range(3)]
    for j in range(_NB):
        l = g * _NB + j
        n = l * _BS + b
        b2 = n // _L
        l2 = n - b2 * _L
        px = pred_ref[b2, l2, 0]
        py = pred_ref[b2, l2, 1]
        fx = ((2.0 * px - 1.0 + 1.0) * _IMG - 1.0) / 2.0
        fy = ((2.0 * py - 1.0 + 1.0) * _IMG - 1.0) / 2.0
        ixn = _round_i32(fx)
        iyn = _round_i32(fy)
        valid = ((ixn >= 0) & (ixn < _IMG) & (iyn >= 0) & (iyn < _IMG))
        vf = valid.astype(jnp.float32)
        ixc = jnp.clip(ixn, 0, _IMG - 1)
        iyc = jnp.clip(iyn, 0, _IMG - 1)
        p = iyc * _IMG + ixc
        pr = p // 128
        pc = p - pr * 128
        sel = (lane == pc).astype(jnp.float32)
        hit = biota == j
        for c in range(3):
            row = img_ref[0, c, pl.ds(pr, 1), :]
            s = jnp.sum(row * sel) * vf
            pooled[c] = jnp.where(hit, s, pooled[c])

    d = (img_ref[0, 0][None] - pooled[0]) ** 2
    d = d + (img_ref[0, 1][None] - pooled[1]) ** 2
    d = d + (img_ref[0, 2][None] - pooled[2]) ** 2

    bits = jax.lax.bitcast_convert_type(d, jnp.int32)

    Ms, As = [], []
    w = bits
    for dpt in range(_D):
        M = jnp.min(w, axis=1, keepdims=True)
        Arow = jnp.min(jnp.where(w == M, rows3, _BIG_I), axis=1,
                       keepdims=True)
        Ms.append(M)
        As.append(Arow)
        if dpt < _D - 1:
            w = jnp.where(rows3 == Arow, _INF_BITS, w)

    Mcur, Acur = Ms[0], As[0]
    dep = jnp.zeros((_NB, 1, 128), jnp.int32)
    badv = jnp.zeros((_NB, 1, 128), jnp.int32)
    xs = jnp.zeros((_NB, 1, _K), jnp.float32)
    ys = jnp.zeros((_NB, 1, _K), jnp.float32)
    for t in range(_K):
        m = jnp.min(Mcur, axis=2, keepdims=True)
        plin = Acur * 128 + lane3
        cand = jnp.where(Mcur == m, plin, _BIG_I)
        pmin = jnp.min(cand, axis=2, keepdims=True)
        popm = cand == pmin
        prow = pmin // _IMG
        pcol = pmin - prow * _IMG
        xk = pcol.astype(jnp.float32) / _IMG
        yk = prow.astype(jnp.float32) / _IMG
        hitk = kio3 == t
        xs = jnp.where(hitk, xk, xs)
        ys = jnp.where(hitk, yk, ys)
        dep = dep + popm.astype(jnp.int32)
        if t < _K - 1:
            badv = jnp.maximum(
                badv, jnp.where(popm & (dep >= _D), 1, 0))
            mnx = jnp.full((_NB, 1, 128), _INF_BITS, jnp.int32)
            anx = jnp.zeros((_NB, 1, 128), jnp.int32)
            for dpt in range(1, _D):
                hitd = dep == dpt
                mnx = jnp.where(hitd, Ms[dpt], mnx)
                anx = jnp.where(hitd, As[dpt], anx)
            Mcur = jnp.where(popm, mnx, Mcur)
            Acur = jnp.where(popm, anx, Acur)
    anybad = jnp.max(badv)

    def _emit(xs_, ys_):
        for j in range(_NB):
            l = g * _NB + j
            xsj = xs_[j]
            ysj = ys_[j]

            @pl.when(l < _L - 1)
            def _():
                qx = pred_ref[b, l + 1, 0]
                qy = pred_ref[b, l + 1, 1]
                dk = (qx - xsj) ** 2 + (qy - ysj) ** 2
                dbest = jnp.min(dk)
                kbest = jnp.min(jnp.where(dk == dbest, kiota, _BIG_I))
                hk = kiota == kbest
                bx = jnp.sum(jnp.where(hk, xsj, 0.0))
                by = jnp.sum(jnp.where(hk, ysj, 0.0))
                out_ref[b, l + 1] = (qx - bx) ** 2 + (qy - by) ** 2

            @pl.when(l == _L - 1)
            def _():
                out_ref[b, 0] = 0.0

    @pl.when(anybad == 0)
    def _():
        _emit(xs, ys)

    @pl.when(anybad != 0)
    def _():
        lin = rows3 * 128 + cols3
        wb = bits
        xs2 = jnp.zeros((_NB, 1, _K), jnp.float32)
        ys2 = jnp.zeros((_NB, 1, _K), jnp.float32)
        for k in range(_K):
            m = jnp.min(jnp.min(wb, axis=1, keepdims=True), axis=2,
                        keepdims=True)
            cand = jnp.where(wb == m, lin, _BIG_I)
            pmin = jnp.min(jnp.min(cand, axis=1, keepdims=True), axis=2,
                           keepdims=True)
            wb = jnp.where(lin == pmin, _INF_BITS, wb)
            prow = pmin // _IMG
            pcol = pmin - prow * _IMG
            xk = pcol.astype(jnp.float32) / _IMG
            yk = prow.astype(jnp.float32) / _IMG
            hitk = kio3 == k
            xs2 = jnp.where(hitk, xk, xs2)
            ys2 = jnp.where(hitk, yk, ys2)
        _emit(xs2, ys2)


@jax.jit
def kernel(predictions, ref_imgs):
    imgs2 = ref_imgs.reshape(_BS, 3, _R, 128)
    contrib = pl.pallas_call(
        _ccloss_kernel,
        grid=(_BS, _L // _NB),
        in_specs=[
            pl.BlockSpec(memory_space=pltpu.SMEM),
            pl.BlockSpec((1, 3, _R, 128), lambda b, g: (b, 0, 0, 0)),
        ],
        out_specs=pl.BlockSpec(memory_space=pltpu.SMEM),
        out_shape=jax.ShapeDtypeStruct((_BS, _L), jnp.float32),
    )(predictions, imgs2)
    return jnp.mean(contrib[:, 1:])
